# baseline (speedup 1.0000x reference)
import jax
import jax.numpy as jnp
from jax.experimental import pallas as pl


def _elu(x):
    return jnp.where(x > 0, x, jnp.expm1(x))


def _bn(x, g, b):
    return g[None, :, None, None] * x + b[None, :, None, None]


def _conv2d(x, w, b, pad):
    y = jax.lax.conv_general_dilated(x, w, (1, 1), [(pad, pad), (pad, pad)],
                                     dimension_numbers=('NCHW', 'OIHW', 'NCHW'))
    return y + b[None, :, None, None]


def _convT2d(x, w, b, stride, pad, opad):
    w2 = jnp.transpose(jnp.flip(w, (2, 3)), (1, 0, 2, 3))
    kh, kw = w.shape[2], w.shape[3]
    ph, pw = pad
    oph, opw = opad
    y = jax.lax.conv_general_dilated(
        x, w2, (1, 1),
        [(kh - 1 - ph, kh - 1 - ph + oph), (kw - 1 - pw, kw - 1 - pw + opw)],
        lhs_dilation=(stride, stride), dimension_numbers=('NCHW', 'OIHW', 'NCHW'))
    return y + b[None, :, None, None]


def _copy_kernel(x_ref, o_ref):
    o_ref[...] = x_ref[...]


def kernel(x, params):
    p = params
    x = pl.pallas_call(
        _copy_kernel,
        out_shape=jax.ShapeDtypeStruct(x.shape, x.dtype),
    )(x)
    B = x.shape[0]
    cl = jnp.transpose(x, (0, 2, 3, 1))
    top3, idx3 = jax.lax.top_k(cl, 3)
    top3 = jnp.transpose(top3, (0, 3, 1, 2))
    idx3 = jnp.transpose(idx3, (0, 3, 1, 2))
    max_vals = jnp.max(top3, axis=1, keepdims=True)
    max_flat = max_vals.reshape(B, 1, -1)
    _, idx = jax.lax.top_k(max_flat, 128)
    idx_sq = idx[:, 0, :]
    flat_top3 = top3.reshape(B, 3, -1)
    flat_idx3 = idx3.reshape(B, 3, -1).astype(jnp.float32)
    idx_b = jnp.broadcast_to(idx_sq[:, None, :], (B, 3, 128))
    reduced_top3 = jnp.take_along_axis(flat_top3, idx_b, axis=2)
    reduced_idx3 = jnp.take_along_axis(flat_idx3, idx_b, axis=2)
    bg_in = jnp.zeros((B, 128), jnp.float32)
    h = bg_in @ p['lin_w'].T + p['lin_b']
    h = _elu(h).reshape(B, 1, 52, 40)
    h = _convT2d(h, p['ct1_w'], p['ct1_b'], 2, (1, 1), (1, 1))
    h = _convT2d(h, p['ct2_w'], p['ct2_b'], 2, (0, 1), (1, 1))
    h = _elu(_bn(h, p['bn1_g'], p['bn1_b']))
    h = _elu(_bn(_conv2d(h, p['c1_w'], p['c1_b'], 0), p['bn2_g'], p['bn2_b']))
    h = _elu(_bn(_conv2d(h, p['c2_w'], p['c2_b'], 2), p['bn3_g'], p['bn3_b']))
    bg = jax.nn.sigmoid(_conv2d(h, p['c3_w'], p['c3_b'], 2))
    rows = jnp.arange(B)[:, None]

    def scat(vals):
        z = jnp.zeros((B, 210 * 160), jnp.float32)
        z = z.at[rows, idx_sq].set(vals)
        return z.reshape(B, 1, 210, 160)

    rec = [scat(reduced_idx3[:, c]) for c in range(3)] + [scat(reduced_top3[:, c]) for c in range(3)]
    sprites_infos = jnp.concatenate(rec, axis=1)
    s = jax.nn.relu(_bn(_convT2d(sprites_infos, p['d3_w1'], p['d3_b1'], 1, (2, 2), (0, 0)), p['d3_bn1_g'], p['d3_bn1_b']))
    s = jax.nn.relu(_bn(_convT2d(s, p['d3_w2'], p['d3_b2'], 1, (2, 2), (0, 0)), p['d3_bn2_g'], p['d3_bn2_b']))
    sprites = jax.nn.sigmoid(_convT2d(s, p['d3_w3'], p['d3_b3'], 1, (2, 2), (0, 0)))
    merged = jnp.concatenate([bg, sprites], axis=1)
    m = jax.nn.relu(_bn(_convT2d(merged, p['d2_w1'], p['d2_b1'], 1, (2, 2), (0, 0)), p['d2_bn1_g'], p['d2_bn1_b']))
    m = jax.nn.relu(_bn(_convT2d(m, p['d2_w2'], p['d2_b2'], 1, (2, 2), (0, 0)), p['d2_bn2_g'], p['d2_bn2_b']))
    out = jax.nn.sigmoid(_convT2d(m, p['d2_w3'], p['d2_b3'], 1, (2, 2), (0, 0)))
    return out


# speedup vs baseline: 4.5253x; 4.5253x over previous
import functools

import jax
import jax.numpy as jnp
from jax.experimental import pallas as pl

H, W = 210, 160
HP, WP = 214, 164
M = HP * WP
TILE = 2304
NT = 16
MARG = TILE
MM = MARG + NT * TILE + MARG
_NEG = -3.0e38


def _elu(x):
    return jnp.where(x > 0, x, jnp.exp(jnp.minimum(x, 0.0)) - 1.0)


def _sigmoid(x):
    return jax.nn.sigmoid(x)


def _relu(x):
    return jnp.maximum(x, 0.0)


def _keep(y, rmask_ref, ch):
    keep = rmask_ref[0:1, :] > 0
    if ch is not None:
        ci = jax.lax.broadcasted_iota(jnp.int32, (y.shape[0], 1), 0)
        keep = keep & (ci >= ch[0]) & (ci < ch[1])
    return jnp.where(keep, y, 0.0)


def _taps(x_ref, t, cin):
    tbase = pl.multiple_of(t * TILE, 128)
    chunks = []
    for dy in range(5):
        for dx in range(5):
            off = MARG + (dy - 2) * WP + (dx - 2)
            off_al = (off // 128) * 128
            r = off - off_al
            wide = x_ref[0, :, pl.ds(tbase + off_al, TILE + 128)]
            chunks.append(jax.lax.slice(wide, (0, r), (cin, r + TILE)))
    return jnp.concatenate(chunks, axis=0)


def _conv5_body(x_ref, w_ref, aff_ref, rmask_ref, o_ref, *, act, odt, ch=None):
    t = pl.program_id(1)
    cin = x_ref.shape[1]
    lhs = _taps(x_ref, t, cin)
    g = jnp.dot(w_ref[...], lhs, preferred_element_type=jnp.float32)
    y = act(g * aff_ref[:, 0:1] + aff_ref[:, 1:2])
    o_ref[0] = _keep(y, rmask_ref, ch).astype(odt)


def _conv5_body2(x_ref, x2_ref, w_ref, aff_ref, rmask_ref, o_ref, *, act, odt, ch=None):
    t = pl.program_id(1)
    cin = x_ref.shape[1]
    lhs = _taps(x_ref, t, cin) + _taps(x2_ref, t, cin)
    g = jnp.dot(w_ref[...], lhs, preferred_element_type=jnp.float32)
    y = act(g * aff_ref[:, 0:1] + aff_ref[:, 1:2])
    o_ref[0] = _keep(y, rmask_ref, ch).astype(odt)


def _conv1_body(x_ref, w_ref, aff_ref, rmask_ref, o_ref, *, act, odt, ch=None):
    t = pl.program_id(1)
    cin = x_ref.shape[1]
    base = pl.multiple_of(t * TILE, 128) + MARG
    lhs = x_ref[0, :, pl.ds(base, TILE)]
    y = jnp.dot(w_ref[...], lhs, preferred_element_type=jnp.float32)
    y = act(y * aff_ref[:, 0:1] + aff_ref[:, 1:2])
    o_ref[0] = _keep(y, rmask_ref, ch).astype(odt)


def _layer(body, x, weights, rmask, cout, odt, x2=None, interpret=False):
    B = x.shape[0]
    ins = [x] + ([x2] if x2 is not None else []) + list(weights) + [rmask]
    specs = [pl.BlockSpec((1, x.shape[1], MM), lambda b, t: (b, 0, 0))]
    if x2 is not None:
        specs.append(pl.BlockSpec((1, x2.shape[1], MM), lambda b, t: (0, 0, 0)))
    for a in weights:
        specs.append(pl.BlockSpec(a.shape, lambda b, t, n=a.ndim: (0,) * n))
    specs.append(pl.BlockSpec((1, TILE), lambda b, t: (0, t + 1)))
    return pl.pallas_call(
        body,
        grid=(B, NT),
        in_specs=specs,
        out_specs=pl.BlockSpec((1, cout, TILE), lambda b, t: (b, 0, t + 1)),
        out_shape=jax.ShapeDtypeStruct((B, cout, MM), odt),
        interpret=interpret,
    )(*ins)


def _topk_body(x_ref, o_ref):
    xb = x_ref[0]
    C = xb.shape[0]
    cio = jax.lax.broadcasted_iota(jnp.int32, (C, H, W), 0)

    m1 = jnp.max(xb, axis=0)
    i1 = jnp.min(jnp.where(xb == m1[None], cio, 99), axis=0)
    x2 = jnp.where(cio == i1[None], _NEG, xb)
    m2 = jnp.max(x2, axis=0)
    i2 = jnp.min(jnp.where(x2 == m2[None], cio, 99), axis=0)
    x3 = jnp.where(cio == i2[None], _NEG, x2)
    m3 = jnp.max(x3, axis=0)
    i3 = jnp.min(jnp.where(x3 == m3[None], cio, 99), axis=0)

    bits = jax.lax.bitcast_convert_type(m1, jnp.int32)
    u = jax.lax.bitcast_convert_type(bits, jnp.uint32)
    key = jnp.where(bits < 0, ~u, u | jnp.uint32(0x80000000))

    K = 128

    def body(_, carry):
        lo, hi = carry
        mid = lo + (hi - lo) // jnp.uint32(2)
        cnt = jnp.sum((key >= mid).astype(jnp.int32))
        big = cnt >= K
        return jnp.where(big, mid, lo), jnp.where(big, hi, mid)

    lo, _ = jax.lax.fori_loop(0, 32, body, (jnp.uint32(0), jnp.uint32(0xFFFFFFFF)))
    t = lo
    cnt_gt = jnp.sum((key > t).astype(jnp.int32))
    m_need = (K - cnt_gt).astype(jnp.float32)

    eq = (key == t).astype(jnp.float32)
    wi = jax.lax.broadcasted_iota(jnp.int32, (W, W), 0)
    wj = jax.lax.broadcasted_iota(jnp.int32, (W, W), 1)
    uw = (wi < wj).astype(jnp.float32)
    prefix_w = jnp.dot(eq, uw, preferred_element_type=jnp.float32)
    rs = jnp.sum(eq, axis=1)[None, :]
    hi_i = jax.lax.broadcasted_iota(jnp.int32, (H, H), 0)
    hj_i = jax.lax.broadcasted_iota(jnp.int32, (H, H), 1)
    uh = (hi_i < hj_i).astype(jnp.float32)
    prefix_h = jnp.dot(rs, uh, preferred_element_type=jnp.float32)
    prefix = prefix_h.reshape(H, 1) + prefix_w
    keep = (key > t) | ((key == t) & (prefix < m_need))
    mask = keep.astype(jnp.float32)

    planes = [mask * i1.astype(jnp.float32), mask * i2.astype(jnp.float32),
              mask * i3.astype(jnp.float32), mask * m1, mask * m2, mask * m3,
              jnp.zeros_like(mask), jnp.zeros_like(mask)]
    stacked = jnp.stack(planes, axis=0)
    o_ref[0] = jnp.pad(stacked, ((0, 0), (2, 2), (2, 2)))


def _convT_3x3_up2(v, w3, b, ph, pw, hout, wout):
    h, w = v.shape
    z = jnp.zeros_like(v)
    vw = jnp.stack([v, z], axis=-1).reshape(h, 2 * w)
    vd = jnp.stack([vw, jnp.zeros_like(vw)], axis=1).reshape(2 * h, 2 * w)
    vp = jnp.pad(vd, (ph, pw))
    out = jnp.zeros((hout, wout), jnp.float32)
    for ky in range(3):
        for kx in range(3):
            out = out + w3[ky, kx] * jax.lax.slice(vp, (ky, kx), (ky + hout, kx + wout))
    return out + b


def _bgseed_body(h0_ref, ct1w_ref, ct1b_ref, ct2w_ref, ct2b_ref, bn1_ref, o_ref):
    h0 = _elu(h0_ref[...])
    h = _convT_3x3_up2(h0, ct1w_ref[...], ct1b_ref[0, 0], (1, 1), (1, 1), 104, 80)
    h = _convT_3x3_up2(h, ct2w_ref[...], ct2b_ref[0, 0], (2, 2), (1, 1), 210, 160)
    h = _elu(bn1_ref[0, 0] * h + bn1_ref[0, 1])
    o_ref[...] = jnp.pad(h, ((2, 2), (2, 2)))


def _wT_from_convT(w, cin_pad, cout_pad, co_off=0):
    w2 = jnp.transpose(jnp.flip(w, (2, 3)), (1, 0, 2, 3))
    wt = jnp.transpose(w2, (0, 2, 3, 1))
    i_dim, o_dim = w.shape[0], w.shape[1]
    full = jnp.zeros((cout_pad, 5, 5, cin_pad), jnp.float32)
    full = full.at[co_off:co_off + o_dim, :, :, :i_dim].set(wt)
    return full.reshape(cout_pad, 25 * cin_pad)


def _wT_from_conv(w, cin_pad, cout_pad, co_off=0):
    wt = jnp.transpose(w, (0, 2, 3, 1))
    o_dim, i_dim = w.shape[0], w.shape[1]
    full = jnp.zeros((cout_pad, 5, 5, cin_pad), jnp.float32)
    full = full.at[co_off:co_off + o_dim, :, :, :i_dim].set(wt)
    return full.reshape(cout_pad, 25 * cin_pad)


def _aff(g, bconv, bbn, cout_pad, co_off=0):
    n = g.shape[0]
    a = jnp.zeros((cout_pad, 2), jnp.float32)
    a = a.at[co_off:co_off + n, 0].set(g)
    a = a.at[co_off:co_off + n, 1].set(g * bconv + bbn)
    return a


def _aff_plain(b, cout_pad, co_off=0):
    n = b.shape[0]
    a = jnp.zeros((cout_pad, 2), jnp.float32)
    a = a.at[co_off:co_off + n, 0].set(jnp.ones((n,), jnp.float32))
    a = a.at[co_off:co_off + n, 1].set(b)
    return a


def _row_mask():
    yi = jax.lax.broadcasted_iota(jnp.int32, (HP, WP), 0)
    xi = jax.lax.broadcasted_iota(jnp.int32, (HP, WP), 1)
    inb = (yi >= 2) & (yi < 2 + H) & (xi >= 2) & (xi < 2 + W)
    m = inb.astype(jnp.float32).reshape(1, M)
    return jnp.pad(m, ((0, 0), (MARG, MM - MARG - M)))


def kernel(x, params, interpret=False):
    p = params
    B = x.shape[0]
    f32 = jnp.float32
    rmask = _row_mask()

    maps = pl.pallas_call(
        _topk_body,
        grid=(B,),
        in_specs=[pl.BlockSpec((1, 16, H, W), lambda b: (b, 0, 0, 0))],
        out_specs=pl.BlockSpec((1, 8, HP, WP), lambda b: (b, 0, 0, 0)),
        out_shape=jax.ShapeDtypeStruct((B, 8, HP, WP), f32),
        interpret=interpret,
    )(x)
    maps = jnp.pad(maps.reshape(B, 8, M), ((0, 0), (0, 0), (MARG, MM - MARG - M)))

    bn1 = jnp.stack([p['bn1_g'], p['bn1_b']], axis=-1).reshape(1, 2)
    bgseed = pl.pallas_call(
        _bgseed_body,
        out_shape=jax.ShapeDtypeStruct((HP, WP), f32),
        interpret=interpret,
    )(p['lin_b'].reshape(52, 40), jnp.flip(p['ct1_w'][0, 0], (0, 1)),
      p['ct1_b'].reshape(1, 1), jnp.flip(p['ct2_w'][0, 0], (0, 1)),
      p['ct2_b'].reshape(1, 1), bn1)
    bgseed = jnp.pad(bgseed.reshape(1, 1, M),
                     ((0, 0), (0, 7), (MARG, MM - MARG - M)))

    c1w = jnp.zeros((64, 8), f32).at[:, 0].set(p['c1_w'][:, 0, 0, 0])
    c1a = _aff(p['bn2_g'], p['c1_b'], p['bn2_b'], 64)
    c2w = _wT_from_conv(p['c2_w'], 64, 64)
    c2a = _aff(p['bn3_g'], p['c2_b'], p['bn3_b'], 64)
    c3w = _wT_from_conv(p['c3_w'], 64, 8)
    c3a = _aff_plain(p['c3_b'], 8)

    b1 = functools.partial(_conv1_body, act=_elu, odt=f32)
    c5 = lambda act, ch=None: functools.partial(_conv5_body, act=act, odt=f32, ch=ch)
    b2 = functools.partial(_conv5_body2, act=_relu, odt=f32)

    a_ = _layer(b1, bgseed, (c1w, c1a), rmask, 64, f32, interpret=interpret)
    a_ = _layer(c5(_elu), a_, (c2w, c2a), rmask, 64, f32, interpret=interpret)
    bg = _layer(c5(_sigmoid, (0, 3)), a_, (c3w, c3a), rmask, 8, f32, interpret=interpret)

    d3w1 = _wT_from_convT(p['d3_w1'], 8, 64)
    d3a1 = _aff(p['d3_bn1_g'], p['d3_b1'], p['d3_bn1_b'], 64)
    d3w2 = _wT_from_convT(p['d3_w2'], 64, 64)
    d3a2 = _aff(p['d3_bn2_g'], p['d3_b2'], p['d3_bn2_b'], 64)
    d3w3 = _wT_from_convT(p['d3_w3'], 64, 8, co_off=3)
    d3a3 = _aff_plain(p['d3_b3'], 8, co_off=3)

    s = _layer(c5(_relu), maps, (d3w1, d3a1), rmask, 64, f32, interpret=interpret)
    s = _layer(c5(_relu), s, (d3w2, d3a2), rmask, 64, f32, interpret=interpret)
    s8 = _layer(c5(_sigmoid, (3, 6)), s, (d3w3, d3a3), rmask, 8, f32, interpret=interpret)

    d2w1 = _wT_from_convT(p['d2_w1'], 8, 64)
    d2a1 = _aff(p['d2_bn1_g'], p['d2_b1'], p['d2_bn1_b'], 64)
    d2w2 = _wT_from_convT(p['d2_w2'], 64, 64)
    d2a2 = _aff(p['d2_bn2_g'], p['d2_b2'], p['d2_bn2_b'], 64)
    d2w3 = _wT_from_convT(p['d2_w3'], 64, 8)
    d2a3 = _aff_plain(p['d2_b3'], 8)

    m_ = _layer(b2, s8, (d2w1, d2a1), rmask, 64, f32, x2=bg, interpret=interpret)
    m_ = _layer(c5(_relu), m_, (d2w2, d2a2), rmask, 64, f32, interpret=interpret)
    out = _layer(c5(_sigmoid), m_, (d2w3, d2a3), rmask, 8, f32, interpret=interpret)

    out = jax.lax.slice(out, (0, 0, MARG), (B, 3, MARG + M))
    return out.reshape(B, 3, HP, WP)[:, :, 2:2 + H, 2:2 + W]


# speedup vs baseline: 5.5257x; 1.2211x over previous
import functools

import jax
import jax.numpy as jnp
from jax.experimental import pallas as pl

H, W = 210, 160
HP, WP = 214, 164
M = HP * WP
TILE = 2304
NT = 16
MARG = TILE
MM = MARG + NT * TILE + MARG
_NEG = -3.0e38


def _elu(x):
    return jnp.where(x > 0, x, jnp.exp(jnp.minimum(x, 0.0)) - 1.0)


def _sigmoid(x):
    return jax.nn.sigmoid(x)


def _relu(x):
    return jnp.maximum(x, 0.0)


def _keep(y, rmask_ref, ch):
    keep = rmask_ref[0:1, :] > 0
    if ch is not None:
        ci = jax.lax.broadcasted_iota(jnp.int32, (y.shape[0], 1), 0)
        keep = keep & (ci >= ch[0]) & (ci < ch[1])
    return jnp.where(keep, y, 0.0)


def _taps(x_ref, t, cin):
    tbase = pl.multiple_of(t * TILE, 128)
    chunks = []
    for dy in range(5):
        for dx in range(5):
            off = MARG + (dy - 2) * WP + (dx - 2)
            off_al = (off // 128) * 128
            r = off - off_al
            wide = x_ref[0, :, pl.ds(tbase + off_al, TILE + 128)]
            chunks.append(jax.lax.slice(wide, (0, r), (cin, r + TILE)))
    return jnp.concatenate(chunks, axis=0)


def _conv5_body(x_ref, w_ref, aff_ref, rmask_ref, o_ref, *, act, odt, ch=None):
    t = pl.program_id(1)
    cin = x_ref.shape[1]
    lhs = _taps(x_ref, t, cin)
    g = jnp.dot(w_ref[...], lhs, preferred_element_type=jnp.float32)
    y = act(g * aff_ref[:, 0:1] + aff_ref[:, 1:2])
    o_ref[0] = _keep(y, rmask_ref, ch).astype(odt)


def _conv5_body2(x_ref, x2_ref, w_ref, aff_ref, rmask_ref, o_ref, *, act, odt, ch=None):
    t = pl.program_id(1)
    cin = x_ref.shape[1]
    lhs = _taps(x_ref, t, cin) + _taps(x2_ref, t, cin)
    g = jnp.dot(w_ref[...], lhs, preferred_element_type=jnp.float32)
    y = act(g * aff_ref[:, 0:1] + aff_ref[:, 1:2])
    o_ref[0] = _keep(y, rmask_ref, ch).astype(odt)


def _conv1_body(x_ref, w_ref, aff_ref, rmask_ref, o_ref, *, act, odt, ch=None):
    t = pl.program_id(1)
    cin = x_ref.shape[1]
    base = pl.multiple_of(t * TILE, 128) + MARG
    lhs = x_ref[0, :, pl.ds(base, TILE)]
    y = jnp.dot(w_ref[...], lhs, preferred_element_type=jnp.float32)
    y = act(y * aff_ref[:, 0:1] + aff_ref[:, 1:2])
    o_ref[0] = _keep(y, rmask_ref, ch).astype(odt)


def _layer(body, x, weights, rmask, cout, odt, x2=None, interpret=False):
    B = x.shape[0]
    ins = [x] + ([x2] if x2 is not None else []) + list(weights) + [rmask]
    specs = [pl.BlockSpec((1, x.shape[1], MM), lambda b, t: (b, 0, 0))]
    if x2 is not None:
        specs.append(pl.BlockSpec((1, x2.shape[1], MM), lambda b, t: (0, 0, 0)))
    for a in weights:
        specs.append(pl.BlockSpec(a.shape, lambda b, t, n=a.ndim: (0,) * n))
    specs.append(pl.BlockSpec((1, TILE), lambda b, t: (0, t + 1)))
    return pl.pallas_call(
        body,
        grid=(B, NT),
        in_specs=specs,
        out_specs=pl.BlockSpec((1, cout, TILE), lambda b, t: (b, 0, t + 1)),
        out_shape=jax.ShapeDtypeStruct((B, cout, MM), odt),
        interpret=interpret,
    )(*ins)


def _topk_body(x_ref, o_ref):
    xb = x_ref[0]
    C = xb.shape[0]
    cio = jax.lax.broadcasted_iota(jnp.int32, (C, H, W), 0)

    m1 = jnp.max(xb, axis=0)
    i1 = jnp.min(jnp.where(xb == m1[None], cio, 99), axis=0)
    x2 = jnp.where(cio == i1[None], _NEG, xb)
    m2 = jnp.max(x2, axis=0)
    i2 = jnp.min(jnp.where(x2 == m2[None], cio, 99), axis=0)
    x3 = jnp.where(cio == i2[None], _NEG, x2)
    m3 = jnp.max(x3, axis=0)
    i3 = jnp.min(jnp.where(x3 == m3[None], cio, 99), axis=0)

    bits = jax.lax.bitcast_convert_type(m1, jnp.int32)
    u = jax.lax.bitcast_convert_type(bits, jnp.uint32)
    key = jnp.where(bits < 0, ~u, u | jnp.uint32(0x80000000))

    K = 128

    def body(_, carry):
        lo, hi = carry
        mid = lo + (hi - lo) // jnp.uint32(2)
        cnt = jnp.sum((key >= mid).astype(jnp.int32))
        big = cnt >= K
        return jnp.where(big, mid, lo), jnp.where(big, hi, mid)

    lo, _ = jax.lax.fori_loop(0, 32, body, (jnp.uint32(0), jnp.uint32(0xFFFFFFFF)))
    t = lo
    cnt_gt = jnp.sum((key > t).astype(jnp.int32))
    m_need = (K - cnt_gt).astype(jnp.float32)

    eq = (key == t).astype(jnp.float32)
    wi = jax.lax.broadcasted_iota(jnp.int32, (W, W), 0)
    wj = jax.lax.broadcasted_iota(jnp.int32, (W, W), 1)
    uw = (wi < wj).astype(jnp.float32)
    prefix_w = jnp.dot(eq, uw, preferred_element_type=jnp.float32)
    rs = jnp.sum(eq, axis=1)[None, :]
    hi_i = jax.lax.broadcasted_iota(jnp.int32, (H, H), 0)
    hj_i = jax.lax.broadcasted_iota(jnp.int32, (H, H), 1)
    uh = (hi_i < hj_i).astype(jnp.float32)
    prefix_h = jnp.dot(rs, uh, preferred_element_type=jnp.float32)
    prefix = prefix_h.reshape(H, 1) + prefix_w
    keep = (key > t) | ((key == t) & (prefix < m_need))
    mask = keep.astype(jnp.float32)

    planes = [mask * i1.astype(jnp.float32), mask * i2.astype(jnp.float32),
              mask * i3.astype(jnp.float32), mask * m1, mask * m2, mask * m3,
              jnp.zeros_like(mask), jnp.zeros_like(mask)]
    stacked = jnp.stack(planes, axis=0)
    o_ref[0] = jnp.pad(stacked, ((0, 0), (2, 2), (2, 2))).astype(jnp.bfloat16)


def _convT_3x3_up2(v, w3, b, ph, pw, hout, wout):
    h, w = v.shape
    z = jnp.zeros_like(v)
    vw = jnp.stack([v, z], axis=-1).reshape(h, 2 * w)
    vd = jnp.stack([vw, jnp.zeros_like(vw)], axis=1).reshape(2 * h, 2 * w)
    vp = jnp.pad(vd, (ph, pw))
    out = jnp.zeros((hout, wout), jnp.float32)
    for ky in range(3):
        for kx in range(3):
            out = out + w3[ky, kx] * jax.lax.slice(vp, (ky, kx), (ky + hout, kx + wout))
    return out + b


def _bgseed_body(h0_ref, ct1w_ref, ct1b_ref, ct2w_ref, ct2b_ref, bn1_ref, o_ref):
    h0 = _elu(h0_ref[...])
    h = _convT_3x3_up2(h0, ct1w_ref[...], ct1b_ref[0, 0], (1, 1), (1, 1), 104, 80)
    h = _convT_3x3_up2(h, ct2w_ref[...], ct2b_ref[0, 0], (2, 2), (1, 1), 210, 160)
    h = _elu(bn1_ref[0, 0] * h + bn1_ref[0, 1])
    o_ref[...] = jnp.pad(h, ((2, 2), (2, 2))).astype(jnp.bfloat16)


def _wT_from_convT(w, cin_pad, cout_pad, co_off=0):
    w2 = jnp.transpose(jnp.flip(w, (2, 3)), (1, 0, 2, 3))
    wt = jnp.transpose(w2, (0, 2, 3, 1))
    i_dim, o_dim = w.shape[0], w.shape[1]
    full = jnp.zeros((cout_pad, 5, 5, cin_pad), jnp.float32)
    full = full.at[co_off:co_off + o_dim, :, :, :i_dim].set(wt)
    return full.reshape(cout_pad, 25 * cin_pad).astype(jnp.bfloat16)


def _wT_from_conv(w, cin_pad, cout_pad, co_off=0):
    wt = jnp.transpose(w, (0, 2, 3, 1))
    o_dim, i_dim = w.shape[0], w.shape[1]
    full = jnp.zeros((cout_pad, 5, 5, cin_pad), jnp.float32)
    full = full.at[co_off:co_off + o_dim, :, :, :i_dim].set(wt)
    return full.reshape(cout_pad, 25 * cin_pad).astype(jnp.bfloat16)


def _aff(g, bconv, bbn, cout_pad, co_off=0):
    n = g.shape[0]
    a = jnp.zeros((cout_pad, 2), jnp.float32)
    a = a.at[co_off:co_off + n, 0].set(g)
    a = a.at[co_off:co_off + n, 1].set(g * bconv + bbn)
    return a


def _aff_plain(b, cout_pad, co_off=0):
    n = b.shape[0]
    a = jnp.zeros((cout_pad, 2), jnp.float32)
    a = a.at[co_off:co_off + n, 0].set(jnp.ones((n,), jnp.float32))
    a = a.at[co_off:co_off + n, 1].set(b)
    return a


def _row_mask():
    yi = jax.lax.broadcasted_iota(jnp.int32, (HP, WP), 0)
    xi = jax.lax.broadcasted_iota(jnp.int32, (HP, WP), 1)
    inb = (yi >= 2) & (yi < 2 + H) & (xi >= 2) & (xi < 2 + W)
    m = inb.astype(jnp.float32).reshape(1, M)
    return jnp.pad(m, ((0, 0), (MARG, MM - MARG - M)))


def kernel(x, params, interpret=False):
    p = params
    B = x.shape[0]
    f32 = jnp.float32
    rmask = _row_mask()

    maps = pl.pallas_call(
        _topk_body,
        grid=(B,),
        in_specs=[pl.BlockSpec((1, 16, H, W), lambda b: (b, 0, 0, 0))],
        out_specs=pl.BlockSpec((1, 8, HP, WP), lambda b: (b, 0, 0, 0)),
        out_shape=jax.ShapeDtypeStruct((B, 8, HP, WP), jnp.bfloat16),
        interpret=interpret,
    )(x)
    maps = jnp.pad(maps.reshape(B, 8, M), ((0, 0), (0, 0), (MARG, MM - MARG - M)))

    bn1 = jnp.stack([p['bn1_g'], p['bn1_b']], axis=-1).reshape(1, 2)
    bgseed = pl.pallas_call(
        _bgseed_body,
        out_shape=jax.ShapeDtypeStruct((HP, WP), jnp.bfloat16),
        interpret=interpret,
    )(p['lin_b'].reshape(52, 40), jnp.flip(p['ct1_w'][0, 0], (0, 1)),
      p['ct1_b'].reshape(1, 1), jnp.flip(p['ct2_w'][0, 0], (0, 1)),
      p['ct2_b'].reshape(1, 1), bn1)
    bgseed = jnp.pad(bgseed.reshape(1, 1, M),
                     ((0, 0), (0, 7), (MARG, MM - MARG - M)))

    c1w = jnp.zeros((64, 8), f32).at[:, 0].set(p['c1_w'][:, 0, 0, 0]).astype(jnp.bfloat16)
    c1a = _aff(p['bn2_g'], p['c1_b'], p['bn2_b'], 64)
    c2w = _wT_from_conv(p['c2_w'], 64, 64)
    c2a = _aff(p['bn3_g'], p['c2_b'], p['bn3_b'], 64)
    c3w = _wT_from_conv(p['c3_w'], 64, 8)
    c3a = _aff_plain(p['c3_b'], 8)

    bf16 = jnp.bfloat16
    b1 = functools.partial(_conv1_body, act=_elu, odt=bf16)
    c5 = lambda act, ch=None, odt=jnp.bfloat16: functools.partial(_conv5_body, act=act, odt=odt, ch=ch)
    b2 = functools.partial(_conv5_body2, act=_relu, odt=bf16)

    a_ = _layer(b1, bgseed, (c1w, c1a), rmask, 64, bf16, interpret=interpret)
    a_ = _layer(c5(_elu), a_, (c2w, c2a), rmask, 64, bf16, interpret=interpret)
    bg = _layer(c5(_sigmoid, (0, 3)), a_, (c3w, c3a), rmask, 8, bf16, interpret=interpret)

    d3w1 = _wT_from_convT(p['d3_w1'], 8, 64)
    d3a1 = _aff(p['d3_bn1_g'], p['d3_b1'], p['d3_bn1_b'], 64)
    d3w2 = _wT_from_convT(p['d3_w2'], 64, 64)
    d3a2 = _aff(p['d3_bn2_g'], p['d3_b2'], p['d3_bn2_b'], 64)
    d3w3 = _wT_from_convT(p['d3_w3'], 64, 8, co_off=3)
    d3a3 = _aff_plain(p['d3_b3'], 8, co_off=3)

    s = _layer(c5(_relu), maps, (d3w1, d3a1), rmask, 64, bf16, interpret=interpret)
    s = _layer(c5(_relu), s, (d3w2, d3a2), rmask, 64, bf16, interpret=interpret)
    s8 = _layer(c5(_sigmoid, (3, 6)), s, (d3w3, d3a3), rmask, 8, bf16, interpret=interpret)

    d2w1 = _wT_from_convT(p['d2_w1'], 8, 64)
    d2a1 = _aff(p['d2_bn1_g'], p['d2_b1'], p['d2_bn1_b'], 64)
    d2w2 = _wT_from_convT(p['d2_w2'], 64, 64)
    d2a2 = _aff(p['d2_bn2_g'], p['d2_b2'], p['d2_bn2_b'], 64)
    d2w3 = _wT_from_convT(p['d2_w3'], 64, 8)
    d2a3 = _aff_plain(p['d2_b3'], 8)

    m_ = _layer(b2, s8, (d2w1, d2a1), rmask, 64, bf16, x2=bg, interpret=interpret)
    m_ = _layer(c5(_relu), m_, (d2w2, d2a2), rmask, 64, bf16, interpret=interpret)
    out = _layer(c5(_sigmoid, None, f32), m_, (d2w3, d2a3), rmask, 8, f32, interpret=interpret)

    out = jax.lax.slice(out, (0, 0, MARG), (B, 3, MARG + M))
    return out.reshape(B, 3, HP, WP)[:, :, 2:2 + H, 2:2 + W]


# speedup vs baseline: 8.0414x; 1.4553x over previous
import functools

import jax
import jax.numpy as jnp
from jax.experimental import pallas as pl

H, W = 210, 160
HP, WP = 214, 164
WF = 256
M = HP * WF
TILE = 2560
NT = 22
MARG = TILE
MM = MARG + NT * TILE + MARG
_NEG = -3.0e38


def _elu(x):
    return jnp.where(x > 0, x, jnp.exp(jnp.minimum(x, 0.0)) - 1.0)


def _sigmoid(x):
    return jax.nn.sigmoid(x)


def _relu(x):
    return jnp.maximum(x, 0.0)


def _keep(y, rmask_ref, ch):
    keep = rmask_ref[0:1, :] > 0
    if ch is not None:
        ci = jax.lax.broadcasted_iota(jnp.int32, (y.shape[0], 1), 0)
        keep = keep & (ci >= ch[0]) & (ci < ch[1])
    return jnp.where(keep, y, 0.0)


def _taps(x_ref, t, cin):
    tbase = pl.multiple_of(t * TILE, 128)
    wide = x_ref[0, :, pl.ds(tbase + MARG - 2 * WF - 128, TILE + 4 * WF + 256)]
    rots = [jax.lax.slice(wide, (0, 126 + dx), (cin, 126 + dx + TILE + 4 * WF))
            for dx in range(5)]
    chunks = [jax.lax.slice(rots[dx], (0, dy * WF), (cin, dy * WF + TILE))
              for dy in range(5) for dx in range(5)]
    return jnp.concatenate(chunks, axis=0)


def _conv5_body(x_ref, w_ref, aff_ref, rmask_ref, o_ref, *, act, odt, ch=None):
    t = pl.program_id(1)
    cin = x_ref.shape[1]
    lhs = _taps(x_ref, t, cin)
    g = jnp.dot(w_ref[...], lhs, preferred_element_type=jnp.float32)
    y = act(g * aff_ref[:, 0:1] + aff_ref[:, 1:2])
    o_ref[0] = _keep(y, rmask_ref, ch).astype(odt)


def _conv5_body2(x_ref, x2_ref, w_ref, aff_ref, rmask_ref, o_ref, *, act, odt, ch=None):
    t = pl.program_id(1)
    cin = x_ref.shape[1]
    lhs = _taps(x_ref, t, cin) + _taps(x2_ref, t, cin)
    g = jnp.dot(w_ref[...], lhs, preferred_element_type=jnp.float32)
    y = act(g * aff_ref[:, 0:1] + aff_ref[:, 1:2])
    o_ref[0] = _keep(y, rmask_ref, ch).astype(odt)


def _conv1_body(x_ref, w_ref, aff_ref, rmask_ref, o_ref, *, act, odt, ch=None):
    t = pl.program_id(1)
    cin = x_ref.shape[1]
    base = pl.multiple_of(t * TILE, 128) + MARG
    lhs = x_ref[0, :, pl.ds(base, TILE)]
    y = jnp.dot(w_ref[...], lhs, preferred_element_type=jnp.float32)
    y = act(y * aff_ref[:, 0:1] + aff_ref[:, 1:2])
    o_ref[0] = _keep(y, rmask_ref, ch).astype(odt)


def _layer(body, x, weights, rmask, cout, odt, x2=None, interpret=False):
    B = x.shape[0]
    ins = [x] + ([x2] if x2 is not None else []) + list(weights) + [rmask]
    specs = [pl.BlockSpec((1, x.shape[1], MM), lambda b, t: (b, 0, 0))]
    if x2 is not None:
        specs.append(pl.BlockSpec((1, x2.shape[1], MM), lambda b, t: (0, 0, 0)))
    for a in weights:
        specs.append(pl.BlockSpec(a.shape, lambda b, t, n=a.ndim: (0,) * n))
    specs.append(pl.BlockSpec((1, TILE), lambda b, t: (0, t + 1)))
    return pl.pallas_call(
        body,
        grid=(B, NT),
        in_specs=specs,
        out_specs=pl.BlockSpec((1, cout, TILE), lambda b, t: (b, 0, t + 1)),
        out_shape=jax.ShapeDtypeStruct((B, cout, MM), odt),
        interpret=interpret,
    )(*ins)


def _topk_body(x_ref, o_ref):
    xb = x_ref[0]
    C = xb.shape[0]
    cio = jax.lax.broadcasted_iota(jnp.int32, (C, H, W), 0)

    m1 = jnp.max(xb, axis=0)
    i1 = jnp.min(jnp.where(xb == m1[None], cio, 99), axis=0)
    x2 = jnp.where(cio == i1[None], _NEG, xb)
    m2 = jnp.max(x2, axis=0)
    i2 = jnp.min(jnp.where(x2 == m2[None], cio, 99), axis=0)
    x3 = jnp.where(cio == i2[None], _NEG, x2)
    m3 = jnp.max(x3, axis=0)
    i3 = jnp.min(jnp.where(x3 == m3[None], cio, 99), axis=0)

    bits = jax.lax.bitcast_convert_type(m1, jnp.int32)
    u = jax.lax.bitcast_convert_type(bits, jnp.uint32)
    key = jnp.where(bits < 0, ~u, u | jnp.uint32(0x80000000))

    K = 128

    def body(_, carry):
        lo, hi = carry
        mid = lo + (hi - lo) // jnp.uint32(2)
        cnt = jnp.sum((key >= mid).astype(jnp.int32))
        big = cnt >= K
        return jnp.where(big, mid, lo), jnp.where(big, hi, mid)

    lo, _ = jax.lax.fori_loop(0, 32, body, (jnp.uint32(0), jnp.uint32(0xFFFFFFFF)))
    t = lo
    cnt_gt = jnp.sum((key > t).astype(jnp.int32))
    m_need = (K - cnt_gt).astype(jnp.float32)

    eq = (key == t).astype(jnp.float32)
    wi = jax.lax.broadcasted_iota(jnp.int32, (W, W), 0)
    wj = jax.lax.broadcasted_iota(jnp.int32, (W, W), 1)
    uw = (wi < wj).astype(jnp.float32)
    prefix_w = jnp.dot(eq, uw, preferred_element_type=jnp.float32)
    rs = jnp.sum(eq, axis=1)[None, :]
    hi_i = jax.lax.broadcasted_iota(jnp.int32, (H, H), 0)
    hj_i = jax.lax.broadcasted_iota(jnp.int32, (H, H), 1)
    uh = (hi_i < hj_i).astype(jnp.float32)
    prefix_h = jnp.dot(rs, uh, preferred_element_type=jnp.float32)
    prefix = prefix_h.reshape(H, 1) + prefix_w
    keep = (key > t) | ((key == t) & (prefix < m_need))
    mask = keep.astype(jnp.float32)

    planes = [mask * i1.astype(jnp.float32), mask * i2.astype(jnp.float32),
              mask * i3.astype(jnp.float32), mask * m1, mask * m2, mask * m3,
              jnp.zeros_like(mask), jnp.zeros_like(mask)]
    stacked = jnp.stack(planes, axis=0)
    o_ref[0] = jnp.pad(stacked, ((0, 0), (2, 2), (2, 2))).astype(jnp.bfloat16)


def _convT_3x3_up2(v, w3, b, ph, pw, hout, wout):
    h, w = v.shape
    z = jnp.zeros_like(v)
    vw = jnp.stack([v, z], axis=-1).reshape(h, 2 * w)
    vd = jnp.stack([vw, jnp.zeros_like(vw)], axis=1).reshape(2 * h, 2 * w)
    vp = jnp.pad(vd, (ph, pw))
    out = jnp.zeros((hout, wout), jnp.float32)
    for ky in range(3):
        for kx in range(3):
            out = out + w3[ky, kx] * jax.lax.slice(vp, (ky, kx), (ky + hout, kx + wout))
    return out + b


def _bgseed_body(h0_ref, ct1w_ref, ct1b_ref, ct2w_ref, ct2b_ref, bn1_ref, o_ref):
    h0 = _elu(h0_ref[...])
    h = _convT_3x3_up2(h0, ct1w_ref[...], ct1b_ref[0, 0], (1, 1), (1, 1), 104, 80)
    h = _convT_3x3_up2(h, ct2w_ref[...], ct2b_ref[0, 0], (2, 2), (1, 1), 210, 160)
    h = _elu(bn1_ref[0, 0] * h + bn1_ref[0, 1])
    o_ref[...] = jnp.pad(h, ((2, 2), (2, 2))).astype(jnp.bfloat16)


def _wT_from_convT(w, cin_pad, cout_pad, co_off=0):
    w2 = jnp.transpose(jnp.flip(w, (2, 3)), (1, 0, 2, 3))
    wt = jnp.transpose(w2, (0, 2, 3, 1))
    i_dim, o_dim = w.shape[0], w.shape[1]
    full = jnp.zeros((cout_pad, 5, 5, cin_pad), jnp.float32)
    full = full.at[co_off:co_off + o_dim, :, :, :i_dim].set(wt)
    return full.reshape(cout_pad, 25 * cin_pad).astype(jnp.bfloat16)


def _wT_from_conv(w, cin_pad, cout_pad, co_off=0):
    wt = jnp.transpose(w, (0, 2, 3, 1))
    o_dim, i_dim = w.shape[0], w.shape[1]
    full = jnp.zeros((cout_pad, 5, 5, cin_pad), jnp.float32)
    full = full.at[co_off:co_off + o_dim, :, :, :i_dim].set(wt)
    return full.reshape(cout_pad, 25 * cin_pad).astype(jnp.bfloat16)


def _aff(g, bconv, bbn, cout_pad, co_off=0):
    n = g.shape[0]
    a = jnp.zeros((cout_pad, 2), jnp.float32)
    a = a.at[co_off:co_off + n, 0].set(g)
    a = a.at[co_off:co_off + n, 1].set(g * bconv + bbn)
    return a


def _aff_plain(b, cout_pad, co_off=0):
    n = b.shape[0]
    a = jnp.zeros((cout_pad, 2), jnp.float32)
    a = a.at[co_off:co_off + n, 0].set(jnp.ones((n,), jnp.float32))
    a = a.at[co_off:co_off + n, 1].set(b)
    return a


def _row_mask():
    yi = jax.lax.broadcasted_iota(jnp.int32, (HP, WF), 0)
    xi = jax.lax.broadcasted_iota(jnp.int32, (HP, WF), 1)
    inb = (yi >= 2) & (yi < 2 + H) & (xi >= 2) & (xi < 2 + W)
    m = inb.astype(jnp.float32).reshape(1, M)
    return jnp.pad(m, ((0, 0), (MARG, MM - MARG - M)))


def kernel(x, params, interpret=False):
    p = params
    B = x.shape[0]
    f32 = jnp.float32
    rmask = _row_mask()

    maps = pl.pallas_call(
        _topk_body,
        grid=(B,),
        in_specs=[pl.BlockSpec((1, 16, H, W), lambda b: (b, 0, 0, 0))],
        out_specs=pl.BlockSpec((1, 8, HP, WP), lambda b: (b, 0, 0, 0)),
        out_shape=jax.ShapeDtypeStruct((B, 8, HP, WP), jnp.bfloat16),
        interpret=interpret,
    )(x)
    maps = jnp.pad(maps, ((0, 0), (0, 0), (0, 0), (0, WF - WP)))
    maps = jnp.pad(maps.reshape(B, 8, M), ((0, 0), (0, 0), (MARG, MM - MARG - M)))

    bn1 = jnp.stack([p['bn1_g'], p['bn1_b']], axis=-1).reshape(1, 2)
    bgseed = pl.pallas_call(
        _bgseed_body,
        out_shape=jax.ShapeDtypeStruct((HP, WP), jnp.bfloat16),
        interpret=interpret,
    )(p['lin_b'].reshape(52, 40), jnp.flip(p['ct1_w'][0, 0], (0, 1)),
      p['ct1_b'].reshape(1, 1), jnp.flip(p['ct2_w'][0, 0], (0, 1)),
      p['ct2_b'].reshape(1, 1), bn1)
    bgseed = jnp.pad(bgseed, ((0, 0), (0, WF - WP)))
    bgseed = jnp.pad(bgseed.reshape(1, 1, M),
                     ((0, 0), (0, 7), (MARG, MM - MARG - M)))

    c1w = jnp.zeros((64, 8), f32).at[:, 0].set(p['c1_w'][:, 0, 0, 0]).astype(jnp.bfloat16)
    c1a = _aff(p['bn2_g'], p['c1_b'], p['bn2_b'], 64)
    c2w = _wT_from_conv(p['c2_w'], 64, 64)
    c2a = _aff(p['bn3_g'], p['c2_b'], p['bn3_b'], 64)
    c3w = _wT_from_conv(p['c3_w'], 64, 8)
    c3a = _aff_plain(p['c3_b'], 8)

    bf16 = jnp.bfloat16
    b1 = functools.partial(_conv1_body, act=_elu, odt=bf16)
    c5 = lambda act, ch=None, odt=jnp.bfloat16: functools.partial(_conv5_body, act=act, odt=odt, ch=ch)
    b2 = functools.partial(_conv5_body2, act=_relu, odt=bf16)

    a_ = _layer(b1, bgseed, (c1w, c1a), rmask, 64, bf16, interpret=interpret)
    a_ = _layer(c5(_elu), a_, (c2w, c2a), rmask, 64, bf16, interpret=interpret)
    bg = _layer(c5(_sigmoid, (0, 3)), a_, (c3w, c3a), rmask, 8, bf16, interpret=interpret)

    d3w1 = _wT_from_convT(p['d3_w1'], 8, 64)
    d3a1 = _aff(p['d3_bn1_g'], p['d3_b1'], p['d3_bn1_b'], 64)
    d3w2 = _wT_from_convT(p['d3_w2'], 64, 64)
    d3a2 = _aff(p['d3_bn2_g'], p['d3_b2'], p['d3_bn2_b'], 64)
    d3w3 = _wT_from_convT(p['d3_w3'], 64, 8, co_off=3)
    d3a3 = _aff_plain(p['d3_b3'], 8, co_off=3)

    s = _layer(c5(_relu), maps, (d3w1, d3a1), rmask, 64, bf16, interpret=interpret)
    s = _layer(c5(_relu), s, (d3w2, d3a2), rmask, 64, bf16, interpret=interpret)
    s8 = _layer(c5(_sigmoid, (3, 6)), s, (d3w3, d3a3), rmask, 8, bf16, interpret=interpret)

    d2w1 = _wT_from_convT(p['d2_w1'], 8, 64)
    d2a1 = _aff(p['d2_bn1_g'], p['d2_b1'], p['d2_bn1_b'], 64)
    d2w2 = _wT_from_convT(p['d2_w2'], 64, 64)
    d2a2 = _aff(p['d2_bn2_g'], p['d2_b2'], p['d2_bn2_b'], 64)
    d2w3 = _wT_from_convT(p['d2_w3'], 64, 8)
    d2a3 = _aff_plain(p['d2_b3'], 8)

    m_ = _layer(b2, s8, (d2w1, d2a1), rmask, 64, bf16, x2=bg, interpret=interpret)
    m_ = _layer(c5(_relu), m_, (d2w2, d2a2), rmask, 64, bf16, interpret=interpret)
    out = _layer(c5(_sigmoid, None, f32), m_, (d2w3, d2a3), rmask, 8, f32, interpret=interpret)

    out = jax.lax.slice(out, (0, 0, MARG), (B, 3, MARG + M))
    return out.reshape(B, 3, HP, WF)[:, :, 2:2 + H, 2:2 + W]
